# Initial kernel scaffold; baseline (speedup 1.0000x reference)
import jax
import jax.numpy as jnp
from jax import lax
from jax.experimental import pallas as pl
from jax.experimental.pallas import tpu as pltpu
from jax.experimental.pallas import tpu_sc as plsc

N = 10000
E = 320000
G = 64
F = 128
MLP_HID = 1024

NC, NS = 2, 16
NW = NC * NS
CH = 128
ROWS_PW = 79
E_PAD = NW * ROWS_PW * CH
IDX_ROWS = NW * ROWS_PW
ACC_ROWS = N + 8
RPS = N // NS

_HIGH = jax.lax.Precision.HIGHEST

_vec_mesh = plsc.VectorSubcoreMesh(
    core_axis_name="c", subcore_axis_name="s", num_cores=NC, num_subcores=NS
)


def _deg_body(src_hbm, dst_hbm, zeros_hbm, out_hbm, sidx, didx, ones_v,
              sacc, dacc):
    cid = lax.axis_index("c")
    sid = lax.axis_index("s")
    wid = cid * NS + sid

    @pl.loop(0, CH)
    def _(i):
        ones_v[i] = jnp.full((16,), 1.0, jnp.float32)

    pltpu.sync_copy(zeros_hbm.at[pl.ds(sid * RPS, RPS)],
                    sacc.at[pl.ds(sid * RPS, RPS)])
    pltpu.sync_copy(zeros_hbm.at[pl.ds(sid * RPS, RPS)],
                    dacc.at[pl.ds(sid * RPS, RPS)])
    plsc.subcore_barrier()

    base = wid * ROWS_PW
    pltpu.sync_copy(src_hbm.at[pl.ds(base, ROWS_PW)], sidx)
    pltpu.sync_copy(dst_hbm.at[pl.ds(base, ROWS_PW)], didx)

    @pl.loop(0, ROWS_PW)
    def _(j):
        pltpu.sync_copy(ones_v, sacc.at[sidx.at[j]], add=True)
        pltpu.sync_copy(ones_v, dacc.at[didx.at[j]], add=True)

    plsc.subcore_barrier()
    pltpu.sync_copy(sacc.at[pl.ds(sid * RPS, RPS)],
                    out_hbm.at[cid, 0, pl.ds(sid * RPS, RPS)])
    pltpu.sync_copy(dacc.at[pl.ds(sid * RPS, RPS)],
                    out_hbm.at[cid, 1, pl.ds(sid * RPS, RPS)])


def _sc_degrees(src2d, dst2d, zeros16):
    k = pl.kernel(
        _deg_body,
        out_type=jax.ShapeDtypeStruct((NC, 2, N, 16), jnp.float32),
        mesh=_vec_mesh,
        scratch_types=[
            pltpu.VMEM((ROWS_PW, CH), jnp.int32),
            pltpu.VMEM((ROWS_PW, CH), jnp.int32),
            pltpu.VMEM((CH, 16), jnp.float32),
            pltpu.VMEM_SHARED((ACC_ROWS, 16), jnp.float32),
            pltpu.VMEM_SHARED((ACC_ROWS, 16), jnp.float32),
        ],
    )
    return k(src2d, dst2d, zeros16)


def _edge_body(hw_hbm, src_hbm, dst_hbm, zeros_hbm, out_hbm, sidx, didx,
               rows_v, sem, acc):
    cid = lax.axis_index("c")
    sid = lax.axis_index("s")
    wid = cid * NS + sid

    pltpu.sync_copy(zeros_hbm.at[pl.ds(sid * RPS, RPS)],
                    acc.at[pl.ds(sid * RPS, RPS)])
    plsc.subcore_barrier()

    base = wid * ROWS_PW
    pltpu.sync_copy(src_hbm.at[pl.ds(base, ROWS_PW)], sidx)
    pltpu.sync_copy(dst_hbm.at[pl.ds(base, ROWS_PW)], didx)

    @pl.loop(0, ROWS_PW)
    def _(j):
        pltpu.async_copy(hw_hbm.at[sidx.at[j]], rows_v, sem).wait()
        pltpu.sync_copy(rows_v, acc.at[didx.at[j]], add=True)

    plsc.subcore_barrier()
    pltpu.sync_copy(acc.at[pl.ds(sid * RPS, RPS)],
                    out_hbm.at[cid, pl.ds(sid * RPS, RPS)])


def _sc_edge_pass(hw, src2d, dst2d, zeros128):
    k = pl.kernel(
        _edge_body,
        out_type=jax.ShapeDtypeStruct((NC, N, F), jnp.float32),
        mesh=_vec_mesh,
        scratch_types=[
            pltpu.VMEM((ROWS_PW, CH), jnp.int32),
            pltpu.VMEM((ROWS_PW, CH), jnp.int32),
            pltpu.VMEM((CH, F), jnp.float32),
            pltpu.SemaphoreType.DMA,
            pltpu.VMEM_SHARED((ACC_ROWS, F), jnp.float32),
        ],
    )
    return k(hw, src2d, dst2d, zeros128)


_BLK = 2000
_NBLK = N // _BLK


def _pre1_body(x_ref, ds0_ref, ds1_ref, w_ref, o_ref):
    deg = ds0_ref[:, 0:1] + ds1_ref[:, 0:1]
    ns = lax.rsqrt(jnp.maximum(deg, 1.0))
    o_ref[...] = jnp.dot(x_ref[...], w_ref[...], precision=_HIGH) * ns


def _tc_pre1(x, ds0, ds1, W1):
    return pl.pallas_call(
        _pre1_body,
        grid=(_NBLK,),
        in_specs=[
            pl.BlockSpec((_BLK, F), lambda i: (i, 0)),
            pl.BlockSpec((_BLK, 16), lambda i: (i, 0)),
            pl.BlockSpec((_BLK, 16), lambda i: (i, 0)),
            pl.BlockSpec((F, F), lambda i: (0, 0)),
        ],
        out_specs=pl.BlockSpec((_BLK, F), lambda i: (i, 0)),
        out_shape=jax.ShapeDtypeStruct((N, F), jnp.float32),
    )(x, ds0, ds1, W1)


def _pre23_body(p0_ref, p1_ref, dd0_ref, dd1_ref, b_ref, ds0_ref, ds1_ref,
                w_ref, o_ref):
    din = dd0_ref[:, 0:1] + dd1_ref[:, 0:1]
    nd = lax.rsqrt(jnp.maximum(din, 1.0))
    h = jnp.maximum((p0_ref[...] + p1_ref[...]) * nd + b_ref[...], 0.0)
    dout = ds0_ref[:, 0:1] + ds1_ref[:, 0:1]
    ns = lax.rsqrt(jnp.maximum(dout, 1.0))
    o_ref[...] = jnp.dot(h, w_ref[...], precision=_HIGH) * ns


def _tc_pre23(p0, p1, dd0, dd1, b, ds0, ds1, W):
    return pl.pallas_call(
        _pre23_body,
        grid=(_NBLK,),
        in_specs=[
            pl.BlockSpec((_BLK, F), lambda i: (i, 0)),
            pl.BlockSpec((_BLK, F), lambda i: (i, 0)),
            pl.BlockSpec((_BLK, 16), lambda i: (i, 0)),
            pl.BlockSpec((_BLK, 16), lambda i: (i, 0)),
            pl.BlockSpec((1, F), lambda i: (0, 0)),
            pl.BlockSpec((_BLK, 16), lambda i: (i, 0)),
            pl.BlockSpec((_BLK, 16), lambda i: (i, 0)),
            pl.BlockSpec((F, F), lambda i: (0, 0)),
        ],
        out_specs=pl.BlockSpec((_BLK, F), lambda i: (i, 0)),
        out_shape=jax.ShapeDtypeStruct((N, F), jnp.float32),
    )(p0, p1, dd0, dd1, b, ds0, ds1, W)


def _head_body(p0_ref, p1_ref, dd0_ref, dd1_ref, b_ref, seg_ref, gf_ref,
               m1_ref, mb1_ref, m2_ref, mb2_ref, m3_ref, mb3_ref, o_ref,
               acc_ref):
    i = pl.program_id(0)

    @pl.when(i == 0)
    def _():
        acc_ref[...] = jnp.zeros_like(acc_ref)

    din = dd0_ref[:, 0:1] + dd1_ref[:, 0:1]
    nd = lax.rsqrt(jnp.maximum(din, 1.0))
    h = jnp.maximum((p0_ref[...] + p1_ref[...]) * nd + b_ref[...], 0.0)
    gids = lax.broadcasted_iota(jnp.float32, (_BLK, G), 1)
    oh = (seg_ref[...] == gids).astype(jnp.float32)
    acc_ref[...] += lax.dot_general(
        oh, h, (((0,), (0,)), ((), ())), precision=_HIGH)

    @pl.when(i == _NBLK - 1)
    def _():
        ge = acc_ref[...]
        z = jnp.dot(ge, m1_ref[0:F, :], precision=_HIGH)
        z += jnp.dot(gf_ref[...], m1_ref[F:F + 16, :], precision=_HIGH)
        z = jnp.maximum(z + mb1_ref[...], 0.0)
        z = jnp.maximum(
            jnp.dot(z, m2_ref[...], precision=_HIGH) + mb2_ref[...], 0.0)
        o_ref[...] = jnp.dot(z, m3_ref[...], precision=_HIGH) + mb3_ref[...]


def _tc_head(p0, p1, dd0, dd1, b3, seg, gf, M1, Mb1, M2, Mb2, M3, Mb3):
    return pl.pallas_call(
        _head_body,
        grid=(_NBLK,),
        in_specs=[
            pl.BlockSpec((_BLK, F), lambda i: (i, 0)),
            pl.BlockSpec((_BLK, F), lambda i: (i, 0)),
            pl.BlockSpec((_BLK, 16), lambda i: (i, 0)),
            pl.BlockSpec((_BLK, 16), lambda i: (i, 0)),
            pl.BlockSpec((1, F), lambda i: (0, 0)),
            pl.BlockSpec((_BLK, 1), lambda i: (i, 0)),
            pl.BlockSpec((G, 16), lambda i: (0, 0)),
            pl.BlockSpec((F + 16, MLP_HID), lambda i: (0, 0)),
            pl.BlockSpec((1, MLP_HID), lambda i: (0, 0)),
            pl.BlockSpec((MLP_HID, MLP_HID), lambda i: (0, 0)),
            pl.BlockSpec((1, MLP_HID), lambda i: (0, 0)),
            pl.BlockSpec((MLP_HID, 2), lambda i: (0, 0)),
            pl.BlockSpec((1, 2), lambda i: (0, 0)),
        ],
        out_specs=pl.BlockSpec((G, 2), lambda i: (0, 0)),
        out_shape=jax.ShapeDtypeStruct((G, 2), jnp.float32),
        scratch_shapes=[pltpu.VMEM((G, F), jnp.float32)],
    )(p0, p1, dd0, dd1, b3, seg, gf, M1, Mb1, M2, Mb2, M3, Mb3)


def kernel(x, edge_index, node_graph_ids, global_feats, W1, b1, W2, b2,
           W3, b3, M1, Mb1, M2, Mb2, M3, Mb3):
    src = edge_index[0].astype(jnp.int32)
    dst = edge_index[1].astype(jnp.int32)
    seg = node_graph_ids.astype(jnp.float32).reshape(N, 1)

    npad = E_PAD - E
    pad_sink = jnp.full((npad,), N, jnp.int32)
    srcg = jnp.concatenate([src, jnp.zeros((npad,), jnp.int32)])
    srcg = srcg.reshape(IDX_ROWS, CH)
    dstg = jnp.concatenate([dst, pad_sink]).reshape(IDX_ROWS, CH)
    srcd = jnp.concatenate([src, pad_sink]).reshape(IDX_ROWS, CH)

    zeros16 = jnp.zeros((N, 16), jnp.float32)
    zeros128 = jnp.zeros((N, F), jnp.float32)

    degs = _sc_degrees(srcd, dstg, zeros16)
    ds0, ds1 = degs[0, 0], degs[1, 0]
    dd0, dd1 = degs[0, 1], degs[1, 1]

    hw = _tc_pre1(x, ds0, ds1, W1)
    p = _sc_edge_pass(hw, srcg, dstg, zeros128)
    hw = _tc_pre23(p[0], p[1], dd0, dd1, b1.reshape(1, F), ds0, ds1, W2)
    p = _sc_edge_pass(hw, srcg, dstg, zeros128)
    hw = _tc_pre23(p[0], p[1], dd0, dd1, b2.reshape(1, F), ds0, ds1, W3)
    p = _sc_edge_pass(hw, srcg, dstg, zeros128)

    return _tc_head(p[0], p[1], dd0, dd1, b3.reshape(1, F), seg,
                    global_feats, M1, Mb1.reshape(1, MLP_HID), M2,
                    Mb2.reshape(1, MLP_HID), M3, Mb3.reshape(1, 2))


# speedup vs baseline: 2.8363x; 2.8363x over previous
import jax
import jax.numpy as jnp
from jax import lax
from jax.experimental import pallas as pl
from jax.experimental.pallas import tpu as pltpu
from jax.experimental.pallas import tpu_sc as plsc

N = 10000
E = 320000
G = 64
F = 128
MLP_HID = 1024

NC, NS = 2, 16
NW = NC * NS
CH = 128
ROWS_PW = 80
E_PAD = NW * ROWS_PW * CH
IDX_ROWS = NW * ROWS_PW
ACC_ROWS = N + 8
RPS = 632
RPS_LAST = N - (NS - 1) * RPS

_HIGH = jax.lax.Precision.HIGHEST

_mesh_cache = []


def _vec_mesh():
    if not _mesh_cache:
        _mesh_cache.append(plsc.VectorSubcoreMesh(
            core_axis_name="c", subcore_axis_name="s",
            num_cores=NC, num_subcores=NS))
    return _mesh_cache[0]


def _per_sub_copy(sid, mk_src, mk_dst, add=False):
    @pl.when(sid < NS - 1)
    def _():
        start = pl.multiple_of(sid * RPS, 8)
        pltpu.sync_copy(mk_src(start, RPS), mk_dst(start, RPS), add=add)

    @pl.when(sid == NS - 1)
    def _():
        start = (NS - 1) * RPS
        pltpu.sync_copy(mk_src(start, RPS_LAST), mk_dst(start, RPS_LAST),
                        add=add)


def _deg_body(src_hbm, dst_hbm, zeros_hbm, ones_hbm, souts_hbm, douts_hbm,
              sidx, didx, ones_v, acc):
    cid = lax.axis_index("c")
    sid = lax.axis_index("s")
    wid = cid * NS + sid

    pltpu.sync_copy(ones_hbm, ones_v)
    _per_sub_copy(sid, lambda s, n: zeros_hbm.at[pl.ds(s, n)],
                  lambda s, n: acc.at[pl.ds(s, n)])
    plsc.subcore_barrier()

    base = pl.multiple_of(wid * ROWS_PW, 8)
    pltpu.sync_copy(src_hbm.at[pl.ds(base, ROWS_PW)], sidx)
    pltpu.sync_copy(dst_hbm.at[pl.ds(base, ROWS_PW)], didx)

    @pl.loop(0, ROWS_PW)
    def _(j):
        pltpu.sync_copy(ones_v, acc.at[sidx.at[j]], add=True)

    plsc.subcore_barrier()
    _per_sub_copy(sid, lambda s, n: acc.at[pl.ds(s, n)],
                  lambda s, n: souts_hbm.at[cid, pl.ds(s, n)])
    plsc.subcore_barrier()

    _per_sub_copy(sid, lambda s, n: zeros_hbm.at[pl.ds(s, n)],
                  lambda s, n: acc.at[pl.ds(s, n)])
    plsc.subcore_barrier()

    @pl.loop(0, ROWS_PW)
    def _(j):
        pltpu.sync_copy(ones_v, acc.at[didx.at[j]], add=True)

    plsc.subcore_barrier()
    _per_sub_copy(sid, lambda s, n: acc.at[pl.ds(s, n)],
                  lambda s, n: douts_hbm.at[cid, pl.ds(s, n)])


def _sc_degrees(src2d, dst2d, zeros128, ones128):
    k = pl.kernel(
        _deg_body,
        out_type=(jax.ShapeDtypeStruct((NC, N, F), jnp.float32),
                  jax.ShapeDtypeStruct((NC, N, F), jnp.float32)),
        mesh=_vec_mesh(),
        scratch_types=[
            pltpu.VMEM((ROWS_PW, CH), jnp.int32),
            pltpu.VMEM((ROWS_PW, CH), jnp.int32),
            pltpu.VMEM((CH, F), jnp.float32),
            pltpu.VMEM_SHARED((ACC_ROWS, F), jnp.float32),
        ],
    )
    return k(src2d, dst2d, zeros128, ones128)


def _edge_body(hw_hbm, src_hbm, dst_hbm, zeros_hbm, out_hbm, sidx, didx,
               rows_v, sem, acc):
    cid = lax.axis_index("c")
    sid = lax.axis_index("s")
    wid = cid * NS + sid

    _per_sub_copy(sid, lambda s, n: zeros_hbm.at[pl.ds(s, n)],
                  lambda s, n: acc.at[pl.ds(s, n)])
    plsc.subcore_barrier()

    base = pl.multiple_of(wid * ROWS_PW, 8)
    pltpu.sync_copy(src_hbm.at[pl.ds(base, ROWS_PW)], sidx)
    pltpu.sync_copy(dst_hbm.at[pl.ds(base, ROWS_PW)], didx)

    @pl.loop(0, ROWS_PW)
    def _(j):
        pltpu.async_copy(hw_hbm.at[sidx.at[j]], rows_v, sem).wait()
        pltpu.sync_copy(rows_v, acc.at[didx.at[j]], add=True)

    plsc.subcore_barrier()
    _per_sub_copy(sid, lambda s, n: acc.at[pl.ds(s, n)],
                  lambda s, n: out_hbm.at[cid, pl.ds(s, n)])


def _sc_edge_pass(hw, src2d, dst2d, zeros128):
    k = pl.kernel(
        _edge_body,
        out_type=jax.ShapeDtypeStruct((NC, N, F), jnp.float32),
        mesh=_vec_mesh(),
        scratch_types=[
            pltpu.VMEM((ROWS_PW, CH), jnp.int32),
            pltpu.VMEM((ROWS_PW, CH), jnp.int32),
            pltpu.VMEM((CH, F), jnp.float32),
            pltpu.SemaphoreType.DMA,
            pltpu.VMEM_SHARED((ACC_ROWS, F), jnp.float32),
        ],
    )
    return k(hw, src2d, dst2d, zeros128)


_BLK = 2000
_NBLK = N // _BLK


def _pre1_body(x_ref, ds0_ref, ds1_ref, w_ref, o_ref):
    deg = ds0_ref[:, 0:1] + ds1_ref[:, 0:1]
    ns = lax.rsqrt(jnp.maximum(deg, 1.0))
    o_ref[...] = jnp.dot(x_ref[...], w_ref[...], precision=_HIGH) * ns


def _tc_pre1(x, ds0, ds1, W1):
    return pl.pallas_call(
        _pre1_body,
        grid=(_NBLK,),
        in_specs=[
            pl.BlockSpec((_BLK, F), lambda i: (i, 0)),
            pl.BlockSpec((_BLK, F), lambda i: (i, 0)),
            pl.BlockSpec((_BLK, F), lambda i: (i, 0)),
            pl.BlockSpec((F, F), lambda i: (0, 0)),
        ],
        out_specs=pl.BlockSpec((_BLK, F), lambda i: (i, 0)),
        out_shape=jax.ShapeDtypeStruct((N, F), jnp.float32),
    )(x, ds0, ds1, W1)


def _pre23_body(p0_ref, p1_ref, dd0_ref, dd1_ref, b_ref, ds0_ref, ds1_ref,
                w_ref, o_ref):
    din = dd0_ref[:, 0:1] + dd1_ref[:, 0:1]
    nd = lax.rsqrt(jnp.maximum(din, 1.0))
    h = jnp.maximum((p0_ref[...] + p1_ref[...]) * nd + b_ref[...], 0.0)
    dout = ds0_ref[:, 0:1] + ds1_ref[:, 0:1]
    ns = lax.rsqrt(jnp.maximum(dout, 1.0))
    o_ref[...] = jnp.dot(h, w_ref[...], precision=_HIGH) * ns


def _tc_pre23(p0, p1, dd0, dd1, b, ds0, ds1, W):
    return pl.pallas_call(
        _pre23_body,
        grid=(_NBLK,),
        in_specs=[
            pl.BlockSpec((_BLK, F), lambda i: (i, 0)),
            pl.BlockSpec((_BLK, F), lambda i: (i, 0)),
            pl.BlockSpec((_BLK, F), lambda i: (i, 0)),
            pl.BlockSpec((_BLK, F), lambda i: (i, 0)),
            pl.BlockSpec((1, F), lambda i: (0, 0)),
            pl.BlockSpec((_BLK, F), lambda i: (i, 0)),
            pl.BlockSpec((_BLK, F), lambda i: (i, 0)),
            pl.BlockSpec((F, F), lambda i: (0, 0)),
        ],
        out_specs=pl.BlockSpec((_BLK, F), lambda i: (i, 0)),
        out_shape=jax.ShapeDtypeStruct((N, F), jnp.float32),
    )(p0, p1, dd0, dd1, b, ds0, ds1, W)


def _head_body(p0_ref, p1_ref, dd0_ref, dd1_ref, b_ref, seg_ref, gf_ref,
               m1_ref, mb1_ref, m2_ref, mb2_ref, m3_ref, mb3_ref, o_ref,
               acc_ref):
    i = pl.program_id(0)

    @pl.when(i == 0)
    def _():
        acc_ref[...] = jnp.zeros_like(acc_ref)

    din = dd0_ref[:, 0:1] + dd1_ref[:, 0:1]
    nd = lax.rsqrt(jnp.maximum(din, 1.0))
    h = jnp.maximum((p0_ref[...] + p1_ref[...]) * nd + b_ref[...], 0.0)
    gids = lax.broadcasted_iota(jnp.int32, (_BLK, G), 1)
    oh = (seg_ref[...] == gids).astype(jnp.float32)
    acc_ref[...] += lax.dot_general(
        oh, h, (((0,), (0,)), ((), ())), precision=_HIGH)

    @pl.when(i == _NBLK - 1)
    def _():
        ge = acc_ref[...]
        z = jnp.dot(ge, m1_ref[0:F, :], precision=_HIGH)
        z += jnp.dot(gf_ref[...], m1_ref[F:F + 16, :], precision=_HIGH)
        z = jnp.maximum(z + mb1_ref[...], 0.0)
        z = jnp.maximum(
            jnp.dot(z, m2_ref[...], precision=_HIGH) + mb2_ref[...], 0.0)
        o_ref[...] = jnp.dot(z, m3_ref[...], precision=_HIGH) + mb3_ref[...]


def _tc_head(p0, p1, dd0, dd1, b3, seg, gf, M1, Mb1, M2, Mb2, M3, Mb3):
    return pl.pallas_call(
        _head_body,
        grid=(_NBLK,),
        in_specs=[
            pl.BlockSpec((_BLK, F), lambda i: (i, 0)),
            pl.BlockSpec((_BLK, F), lambda i: (i, 0)),
            pl.BlockSpec((_BLK, F), lambda i: (i, 0)),
            pl.BlockSpec((_BLK, F), lambda i: (i, 0)),
            pl.BlockSpec((1, F), lambda i: (0, 0)),
            pl.BlockSpec((_BLK, 1), lambda i: (i, 0)),
            pl.BlockSpec((G, 16), lambda i: (0, 0)),
            pl.BlockSpec((F + 16, MLP_HID), lambda i: (0, 0)),
            pl.BlockSpec((1, MLP_HID), lambda i: (0, 0)),
            pl.BlockSpec((MLP_HID, MLP_HID), lambda i: (0, 0)),
            pl.BlockSpec((1, MLP_HID), lambda i: (0, 0)),
            pl.BlockSpec((MLP_HID, 2), lambda i: (0, 0)),
            pl.BlockSpec((1, 2), lambda i: (0, 0)),
        ],
        out_specs=pl.BlockSpec((G, 2), lambda i: (0, 0)),
        out_shape=jax.ShapeDtypeStruct((G, 2), jnp.float32),
        scratch_shapes=[pltpu.VMEM((G, F), jnp.float32)],
    )(p0, p1, dd0, dd1, b3, seg, gf, M1, Mb1, M2, Mb2, M3, Mb3)


def kernel(x, edge_index, node_graph_ids, global_feats, W1, b1, W2, b2,
           W3, b3, M1, Mb1, M2, Mb2, M3, Mb3):
    src = edge_index[0].astype(jnp.int32)
    dst = edge_index[1].astype(jnp.int32)
    seg = node_graph_ids.astype(jnp.int32).reshape(N, 1)

    npad = E_PAD - E
    pad_sink = jnp.full((npad,), N, jnp.int32)
    srcg = jnp.concatenate([src, jnp.zeros((npad,), jnp.int32)])
    srcg = srcg.reshape(IDX_ROWS, CH)
    dstg = jnp.concatenate([dst, pad_sink]).reshape(IDX_ROWS, CH)
    srcd = jnp.concatenate([src, pad_sink]).reshape(IDX_ROWS, CH)

    zeros128 = jnp.zeros((N, F), jnp.float32)
    ones128 = jnp.ones((CH, F), jnp.float32)
    sdeg, ddeg = _sc_degrees(srcd, dstg, zeros128, ones128)
    ds0, ds1 = sdeg[0], sdeg[1]
    dd0, dd1 = ddeg[0], ddeg[1]

    hw = _tc_pre1(x, ds0, ds1, W1)
    p = _sc_edge_pass(hw, srcg, dstg, zeros128)
    hw = _tc_pre23(p[0], p[1], dd0, dd1, b1.reshape(1, F), ds0, ds1, W2)
    p = _sc_edge_pass(hw, srcg, dstg, zeros128)
    hw = _tc_pre23(p[0], p[1], dd0, dd1, b2.reshape(1, F), ds0, ds1, W3)
    p = _sc_edge_pass(hw, srcg, dstg, zeros128)

    return _tc_head(p[0], p[1], dd0, dd1, b3.reshape(1, F), seg,
                    global_feats, M1, Mb1.reshape(1, MLP_HID), M2,
                    Mb2.reshape(1, MLP_HID), M3, Mb3.reshape(1, 2))


# speedup vs baseline: 5.6334x; 1.9862x over previous
import jax
import jax.numpy as jnp
from jax import lax
from jax.experimental import pallas as pl
from jax.experimental.pallas import tpu as pltpu
from jax.experimental.pallas import tpu_sc as plsc

N = 10000
E = 320000
G = 64
F = 128
FH = F // 2
MLP_HID = 1024

NC, NS = 2, 16
CH = 128
RPSUB = 160
IDXB = 16
IDX_ROWS = NS * RPSUB
E_PAD = IDX_ROWS * CH
PADR = 1024
ACC_ROWS = N + PADR
RPS = 632
RPS_LAST = N - (NS - 1) * RPS

_HIGH = jax.lax.Precision.HIGHEST

_mesh_cache = []


def _vec_mesh():
    if not _mesh_cache:
        _mesh_cache.append(plsc.VectorSubcoreMesh(
            core_axis_name="c", subcore_axis_name="s",
            num_cores=NC, num_subcores=NS))
    return _mesh_cache[0]


def _per_sub_copy(sid, mk_src, mk_dst):
    @pl.when(sid < NS - 1)
    def _():
        start = pl.multiple_of(sid * RPS, 8)
        pltpu.sync_copy(mk_src(start, RPS), mk_dst(start, RPS))

    @pl.when(sid == NS - 1)
    def _():
        start = (NS - 1) * RPS
        pltpu.sync_copy(mk_src(start, RPS_LAST), mk_dst(start, RPS_LAST))


def _deg_body(idx2_hbm, zeros_hbm, ones_hbm, out_hbm, idxs, ones_v, acc):
    cid = lax.axis_index("c")
    sid = lax.axis_index("s")

    pltpu.sync_copy(ones_hbm, ones_v)
    _per_sub_copy(sid, lambda s, n: zeros_hbm.at[pl.ds(s, n)],
                  lambda s, n: acc.at[pl.ds(s, n)])
    plsc.subcore_barrier()

    base = pl.multiple_of(sid * RPSUB, 8)
    pltpu.sync_copy(idx2_hbm.at[cid, pl.ds(base, RPSUB)], idxs)

    @pl.loop(0, RPSUB)
    def _(j):
        pltpu.sync_copy(ones_v, acc.at[idxs.at[j]], add=True)

    plsc.subcore_barrier()
    _per_sub_copy(sid, lambda s, n: acc.at[pl.ds(s, n)],
                  lambda s, n: out_hbm.at[cid, pl.ds(s, n)])


def _sc_degrees(idx2, zeros64, ones64):
    k = pl.kernel(
        _deg_body,
        compiler_params=pltpu.CompilerParams(use_tc_tiling_on_sc=False),
        out_type=jax.ShapeDtypeStruct((NC, N, FH), jnp.float32),
        mesh=_vec_mesh(),
        scratch_types=[
            pltpu.VMEM((RPSUB, CH), jnp.int32),
            pltpu.VMEM((CH, FH), jnp.float32),
            pltpu.VMEM_SHARED((ACC_ROWS, FH), jnp.float32),
        ],
    )
    return k(idx2, zeros64, ones64)


def _edge_body(hw2_hbm, sidx_hbm, didx_hbm, zeros_hbm, out_hbm, sidx, didx,
               b0, b1, sem0, sem1, table, acc):
    cid = lax.axis_index("c")
    sid = lax.axis_index("s")

    _per_sub_copy(sid, lambda s, n: hw2_hbm.at[cid, pl.ds(s, n)],
                  lambda s, n: table.at[pl.ds(s, n)])
    _per_sub_copy(sid, lambda s, n: zeros_hbm.at[pl.ds(s, n)],
                  lambda s, n: acc.at[pl.ds(s, n)])
    plsc.subcore_barrier()

    @pl.loop(0, RPSUB // IDXB)
    def _(k):
        off = pl.multiple_of(sid * RPSUB + k * IDXB, 8)
        pltpu.sync_copy(sidx_hbm.at[pl.ds(off, IDXB)], sidx)
        pltpu.sync_copy(didx_hbm.at[pl.ds(off, IDXB)], didx)

        @pl.loop(0, IDXB // 2)
        def _(jj):
            j0 = jj * 2
            j1 = j0 + 1
            d0 = pltpu.async_copy(table.at[sidx.at[j0]], b0, sem0)
            d1 = pltpu.async_copy(table.at[sidx.at[j1]], b1, sem1)
            d0.wait()
            pltpu.sync_copy(b0, acc.at[didx.at[j0]], add=True)
            d1.wait()
            pltpu.sync_copy(b1, acc.at[didx.at[j1]], add=True)

    plsc.subcore_barrier()
    _per_sub_copy(sid, lambda s, n: acc.at[pl.ds(s, n)],
                  lambda s, n: out_hbm.at[cid, pl.ds(s, n)])


def _sc_edge_pass(hw2, srcg, dstg, zeros64):
    k = pl.kernel(
        _edge_body,
        compiler_params=pltpu.CompilerParams(use_tc_tiling_on_sc=False),
        out_type=jax.ShapeDtypeStruct((NC, N, FH), jnp.float32),
        mesh=_vec_mesh(),
        scratch_types=[
            pltpu.VMEM((IDXB, CH), jnp.int32),
            pltpu.VMEM((IDXB, CH), jnp.int32),
            pltpu.VMEM((CH, FH), jnp.float32),
            pltpu.VMEM((CH, FH), jnp.float32),
            pltpu.SemaphoreType.DMA,
            pltpu.SemaphoreType.DMA,
            pltpu.VMEM_SHARED((N, FH), jnp.float32),
            pltpu.VMEM_SHARED((ACC_ROWS, FH), jnp.float32),
        ],
    )
    return k(hw2, srcg, dstg, zeros64)


_BLK = 2000
_NBLK = N // _BLK


def _pre1_body(x_ref, ds_ref, w_ref, o_ref):
    ns = lax.rsqrt(jnp.maximum(ds_ref[:, 0:1], 1.0))
    y = jnp.dot(x_ref[...], w_ref[...], precision=_HIGH) * ns
    o_ref[0] = y[:, :FH]
    o_ref[1] = y[:, FH:]


def _tc_pre1(x, ds, W1):
    return pl.pallas_call(
        _pre1_body,
        grid=(_NBLK,),
        in_specs=[
            pl.BlockSpec((_BLK, F), lambda i: (i, 0)),
            pl.BlockSpec((_BLK, FH), lambda i: (i, 0)),
            pl.BlockSpec((F, F), lambda i: (0, 0)),
        ],
        out_specs=pl.BlockSpec((NC, _BLK, FH), lambda i: (0, i, 0)),
        out_shape=jax.ShapeDtypeStruct((NC, N, FH), jnp.float32),
    )(x, ds, W1)


def _pre23_body(p_ref, dd_ref, b_ref, ds_ref, w_ref, o_ref):
    nd = lax.rsqrt(jnp.maximum(dd_ref[:, 0:1], 1.0))
    agg = jnp.concatenate([p_ref[0], p_ref[1]], axis=-1)
    h = jnp.maximum(agg * nd + b_ref[...], 0.0)
    ns = lax.rsqrt(jnp.maximum(ds_ref[:, 0:1], 1.0))
    y = jnp.dot(h, w_ref[...], precision=_HIGH) * ns
    o_ref[0] = y[:, :FH]
    o_ref[1] = y[:, FH:]


def _tc_pre23(p, dd, b, ds, W):
    return pl.pallas_call(
        _pre23_body,
        grid=(_NBLK,),
        in_specs=[
            pl.BlockSpec((NC, _BLK, FH), lambda i: (0, i, 0)),
            pl.BlockSpec((_BLK, FH), lambda i: (i, 0)),
            pl.BlockSpec((1, F), lambda i: (0, 0)),
            pl.BlockSpec((_BLK, FH), lambda i: (i, 0)),
            pl.BlockSpec((F, F), lambda i: (0, 0)),
        ],
        out_specs=pl.BlockSpec((NC, _BLK, FH), lambda i: (0, i, 0)),
        out_shape=jax.ShapeDtypeStruct((NC, N, FH), jnp.float32),
    )(p, dd, b, ds, W)


def _head_body(p_ref, dd_ref, b_ref, seg_ref, gf_ref, m1_ref, mb1_ref,
               m2_ref, mb2_ref, m3_ref, mb3_ref, o_ref, acc_ref):
    i = pl.program_id(0)

    @pl.when(i == 0)
    def _():
        acc_ref[...] = jnp.zeros_like(acc_ref)

    nd = lax.rsqrt(jnp.maximum(dd_ref[:, 0:1], 1.0))
    agg = jnp.concatenate([p_ref[0], p_ref[1]], axis=-1)
    h = jnp.maximum(agg * nd + b_ref[...], 0.0)
    gids = lax.broadcasted_iota(jnp.int32, (_BLK, G), 1)
    oh = (seg_ref[...] == gids).astype(jnp.float32)
    acc_ref[...] += lax.dot_general(
        oh, h, (((0,), (0,)), ((), ())), precision=_HIGH)

    @pl.when(i == _NBLK - 1)
    def _():
        ge = acc_ref[...]
        z = jnp.dot(ge, m1_ref[0:F, :], precision=_HIGH)
        z += jnp.dot(gf_ref[...], m1_ref[F:F + 16, :], precision=_HIGH)
        z = jnp.maximum(z + mb1_ref[...], 0.0)
        z = jnp.maximum(
            jnp.dot(z, m2_ref[...], precision=_HIGH) + mb2_ref[...], 0.0)
        o_ref[...] = jnp.dot(z, m3_ref[...], precision=_HIGH) + mb3_ref[...]


def _tc_head(p, dd, b3, seg, gf, M1, Mb1, M2, Mb2, M3, Mb3):
    return pl.pallas_call(
        _head_body,
        grid=(_NBLK,),
        in_specs=[
            pl.BlockSpec((NC, _BLK, FH), lambda i: (0, i, 0)),
            pl.BlockSpec((_BLK, FH), lambda i: (i, 0)),
            pl.BlockSpec((1, F), lambda i: (0, 0)),
            pl.BlockSpec((_BLK, 1), lambda i: (i, 0)),
            pl.BlockSpec((G, 16), lambda i: (0, 0)),
            pl.BlockSpec((F + 16, MLP_HID), lambda i: (0, 0)),
            pl.BlockSpec((1, MLP_HID), lambda i: (0, 0)),
            pl.BlockSpec((MLP_HID, MLP_HID), lambda i: (0, 0)),
            pl.BlockSpec((1, MLP_HID), lambda i: (0, 0)),
            pl.BlockSpec((MLP_HID, 2), lambda i: (0, 0)),
            pl.BlockSpec((1, 2), lambda i: (0, 0)),
        ],
        out_specs=pl.BlockSpec((G, 2), lambda i: (0, 0)),
        out_shape=jax.ShapeDtypeStruct((G, 2), jnp.float32),
        scratch_shapes=[pltpu.VMEM((G, F), jnp.float32)],
    )(p, dd, b3, seg, gf, M1, Mb1, M2, Mb2, M3, Mb3)


def kernel(x, edge_index, node_graph_ids, global_feats, W1, b1, W2, b2,
           W3, b3, M1, Mb1, M2, Mb2, M3, Mb3):
    src = edge_index[0].astype(jnp.int32)
    dst = edge_index[1].astype(jnp.int32)
    seg = node_graph_ids.astype(jnp.int32).reshape(N, 1)

    npad = E_PAD - E
    pad_sink = N + (jnp.arange(npad, dtype=jnp.int32) % PADR)
    srcg = jnp.concatenate([src, jnp.zeros((npad,), jnp.int32)])
    srcg = srcg.reshape(IDX_ROWS, CH)
    dstg = jnp.concatenate([dst, pad_sink]).reshape(IDX_ROWS, CH)
    srcd = jnp.concatenate([src, pad_sink]).reshape(IDX_ROWS, CH)
    idx2 = jnp.stack([srcd, dstg])

    zeros64 = jnp.zeros((N, FH), jnp.float32)
    ones64 = jnp.ones((CH, FH), jnp.float32)

    degs = _sc_degrees(idx2, zeros64, ones64)
    ds, dd = degs[0], degs[1]

    hw = _tc_pre1(x, ds, W1)
    p = _sc_edge_pass(hw, srcg, dstg, zeros64)
    hw = _tc_pre23(p, dd, b1.reshape(1, F), ds, W2)
    p = _sc_edge_pass(hw, srcg, dstg, zeros64)
    hw = _tc_pre23(p, dd, b2.reshape(1, F), ds, W3)
    p = _sc_edge_pass(hw, srcg, dstg, zeros64)

    return _tc_head(p, dd, b3.reshape(1, F), seg, global_feats, M1,
                    Mb1.reshape(1, MLP_HID), M2, Mb2.reshape(1, MLP_HID),
                    M3, Mb3.reshape(1, 2))


# speedup vs baseline: 6.1642x; 1.0942x over previous
import jax
import jax.numpy as jnp
from jax import lax
from jax.experimental import pallas as pl
from jax.experimental.pallas import tpu as pltpu
from jax.experimental.pallas import tpu_sc as plsc

N = 10000
E = 320000
G = 64
F = 128
FH = F // 2
MLP_HID = 1024

NC, NS = 2, 16
CH = 128
RPSUB = 160
IDXB = 16
IDX_ROWS = NS * RPSUB
E_PAD = IDX_ROWS * CH
PADR = 512
ACC_ROWS = N + PADR
RPS = 632
RPS_LAST = N - (NS - 1) * RPS

_HIGH = jax.lax.Precision.HIGHEST

_mesh_cache = []


def _vec_mesh():
    if not _mesh_cache:
        _mesh_cache.append(plsc.VectorSubcoreMesh(
            core_axis_name="c", subcore_axis_name="s",
            num_cores=NC, num_subcores=NS))
    return _mesh_cache[0]


def _per_sub_copy(sid, mk_src, mk_dst):
    @pl.when(sid < NS - 1)
    def _():
        start = pl.multiple_of(sid * RPS, 8)
        pltpu.sync_copy(mk_src(start, RPS), mk_dst(start, RPS))

    @pl.when(sid == NS - 1)
    def _():
        start = (NS - 1) * RPS
        pltpu.sync_copy(mk_src(start, RPS_LAST), mk_dst(start, RPS_LAST))


def _deg_body(idx2_hbm, zeros_hbm, ones_hbm, out_hbm, idxs, ones_v, acc):
    cid = lax.axis_index("c")
    sid = lax.axis_index("s")

    pltpu.sync_copy(ones_hbm, ones_v)
    _per_sub_copy(sid, lambda s, n: zeros_hbm.at[pl.ds(s, n)],
                  lambda s, n: acc.at[pl.ds(s, n)])
    plsc.subcore_barrier()

    base = pl.multiple_of(sid * RPSUB, 8)
    pltpu.sync_copy(idx2_hbm.at[cid, pl.ds(base, RPSUB)], idxs)

    @pl.loop(0, RPSUB)
    def _(j):
        pltpu.sync_copy(ones_v, acc.at[idxs.at[j]], add=True)

    plsc.subcore_barrier()
    _per_sub_copy(sid, lambda s, n: acc.at[pl.ds(s, n)],
                  lambda s, n: out_hbm.at[cid, pl.ds(s, n)])


def _sc_degrees(idx2, zeros64, ones64):
    k = pl.kernel(
        _deg_body,
        compiler_params=pltpu.CompilerParams(use_tc_tiling_on_sc=False),
        out_type=jax.ShapeDtypeStruct((NC, N, FH), jnp.float32),
        mesh=_vec_mesh(),
        scratch_types=[
            pltpu.VMEM((RPSUB, CH), jnp.int32),
            pltpu.VMEM((CH, FH), jnp.float32),
            pltpu.VMEM_SHARED((ACC_ROWS, FH), jnp.float32),
        ],
    )
    return k(idx2, zeros64, ones64)


def _edge_body(hw2_hbm, sidx_hbm, didx_hbm, zeros_hbm, out_hbm, sidx, didx,
               b0, b1, b2, b3, sem0, sem1, sem2, sem3, table, acc):
    cid = lax.axis_index("c")
    sid = lax.axis_index("s")

    _per_sub_copy(sid, lambda s, n: hw2_hbm.at[cid, pl.ds(s, n)],
                  lambda s, n: table.at[pl.ds(s, n)])
    _per_sub_copy(sid, lambda s, n: zeros_hbm.at[pl.ds(s, n)],
                  lambda s, n: acc.at[pl.ds(s, n)])
    plsc.subcore_barrier()

    @pl.loop(0, RPSUB // IDXB)
    def _(k):
        off = pl.multiple_of(sid * RPSUB + k * IDXB, 8)
        pltpu.sync_copy(sidx_hbm.at[pl.ds(off, IDXB)], sidx)
        pltpu.sync_copy(didx_hbm.at[pl.ds(off, IDXB)], didx)

        @pl.loop(0, IDXB // 4)
        def _(g):
            j0 = g * 4
            d0 = pltpu.async_copy(table.at[sidx.at[j0]], b0, sem0)
            d1 = pltpu.async_copy(table.at[sidx.at[j0 + 1]], b1, sem1)
            d0.wait()
            d2 = pltpu.async_copy(table.at[sidx.at[j0 + 2]], b2, sem2)
            pltpu.sync_copy(b0, acc.at[didx.at[j0]], add=True)
            d1.wait()
            d3 = pltpu.async_copy(table.at[sidx.at[j0 + 3]], b3, sem3)
            pltpu.sync_copy(b1, acc.at[didx.at[j0 + 1]], add=True)
            d2.wait()
            pltpu.sync_copy(b2, acc.at[didx.at[j0 + 2]], add=True)
            d3.wait()
            pltpu.sync_copy(b3, acc.at[didx.at[j0 + 3]], add=True)

    plsc.subcore_barrier()
    _per_sub_copy(sid, lambda s, n: acc.at[pl.ds(s, n)],
                  lambda s, n: out_hbm.at[cid, pl.ds(s, n)])


def _sc_edge_pass(hw2, srcg, dstg, zeros64):
    k = pl.kernel(
        _edge_body,
        compiler_params=pltpu.CompilerParams(use_tc_tiling_on_sc=False),
        out_type=jax.ShapeDtypeStruct((NC, N, FH), jnp.float32),
        mesh=_vec_mesh(),
        scratch_types=[
            pltpu.VMEM((IDXB, CH), jnp.int32),
            pltpu.VMEM((IDXB, CH), jnp.int32),
            pltpu.VMEM((CH, FH), jnp.float32),
            pltpu.VMEM((CH, FH), jnp.float32),
            pltpu.VMEM((CH, FH), jnp.float32),
            pltpu.VMEM((CH, FH), jnp.float32),
            pltpu.SemaphoreType.DMA,
            pltpu.SemaphoreType.DMA,
            pltpu.SemaphoreType.DMA,
            pltpu.SemaphoreType.DMA,
            pltpu.VMEM_SHARED((N, FH), jnp.float32),
            pltpu.VMEM_SHARED((ACC_ROWS, FH), jnp.float32),
        ],
    )
    return k(hw2, srcg, dstg, zeros64)


_BLK = 2000
_NBLK = N // _BLK


def _pre1_body(x_ref, ds_ref, w_ref, o_ref):
    ns = lax.rsqrt(jnp.maximum(ds_ref[:, 0:1], 1.0))
    y = jnp.dot(x_ref[...], w_ref[...], precision=_HIGH) * ns
    o_ref[0] = y[:, :FH]
    o_ref[1] = y[:, FH:]


def _tc_pre1(x, ds, W1):
    return pl.pallas_call(
        _pre1_body,
        grid=(_NBLK,),
        in_specs=[
            pl.BlockSpec((_BLK, F), lambda i: (i, 0)),
            pl.BlockSpec((_BLK, FH), lambda i: (i, 0)),
            pl.BlockSpec((F, F), lambda i: (0, 0)),
        ],
        out_specs=pl.BlockSpec((NC, _BLK, FH), lambda i: (0, i, 0)),
        out_shape=jax.ShapeDtypeStruct((NC, N, FH), jnp.float32),
    )(x, ds, W1)


def _pre23_body(p_ref, dd_ref, b_ref, ds_ref, w_ref, o_ref):
    nd = lax.rsqrt(jnp.maximum(dd_ref[:, 0:1], 1.0))
    agg = jnp.concatenate([p_ref[0], p_ref[1]], axis=-1)
    h = jnp.maximum(agg * nd + b_ref[...], 0.0)
    ns = lax.rsqrt(jnp.maximum(ds_ref[:, 0:1], 1.0))
    y = jnp.dot(h, w_ref[...], precision=_HIGH) * ns
    o_ref[0] = y[:, :FH]
    o_ref[1] = y[:, FH:]


def _tc_pre23(p, dd, b, ds, W):
    return pl.pallas_call(
        _pre23_body,
        grid=(_NBLK,),
        in_specs=[
            pl.BlockSpec((NC, _BLK, FH), lambda i: (0, i, 0)),
            pl.BlockSpec((_BLK, FH), lambda i: (i, 0)),
            pl.BlockSpec((1, F), lambda i: (0, 0)),
            pl.BlockSpec((_BLK, FH), lambda i: (i, 0)),
            pl.BlockSpec((F, F), lambda i: (0, 0)),
        ],
        out_specs=pl.BlockSpec((NC, _BLK, FH), lambda i: (0, i, 0)),
        out_shape=jax.ShapeDtypeStruct((NC, N, FH), jnp.float32),
    )(p, dd, b, ds, W)


def _head_body(p_ref, dd_ref, b_ref, seg_ref, gf_ref, m1_ref, mb1_ref,
               m2_ref, mb2_ref, m3_ref, mb3_ref, o_ref, acc_ref):
    i = pl.program_id(0)

    @pl.when(i == 0)
    def _():
        acc_ref[...] = jnp.zeros_like(acc_ref)

    nd = lax.rsqrt(jnp.maximum(dd_ref[:, 0:1], 1.0))
    agg = jnp.concatenate([p_ref[0], p_ref[1]], axis=-1)
    h = jnp.maximum(agg * nd + b_ref[...], 0.0)
    gids = lax.broadcasted_iota(jnp.int32, (_BLK, G), 1)
    oh = (seg_ref[...] == gids).astype(jnp.float32)
    acc_ref[...] += lax.dot_general(
        oh, h, (((0,), (0,)), ((), ())), precision=_HIGH)

    @pl.when(i == _NBLK - 1)
    def _():
        ge = acc_ref[...]
        z = jnp.dot(ge, m1_ref[0:F, :], precision=_HIGH)
        z += jnp.dot(gf_ref[...], m1_ref[F:F + 16, :], precision=_HIGH)
        z = jnp.maximum(z + mb1_ref[...], 0.0)
        z = jnp.maximum(
            jnp.dot(z, m2_ref[...], precision=_HIGH) + mb2_ref[...], 0.0)
        o_ref[...] = jnp.dot(z, m3_ref[...], precision=_HIGH) + mb3_ref[...]


def _tc_head(p, dd, b3, seg, gf, M1, Mb1, M2, Mb2, M3, Mb3):
    return pl.pallas_call(
        _head_body,
        grid=(_NBLK,),
        in_specs=[
            pl.BlockSpec((NC, _BLK, FH), lambda i: (0, i, 0)),
            pl.BlockSpec((_BLK, FH), lambda i: (i, 0)),
            pl.BlockSpec((1, F), lambda i: (0, 0)),
            pl.BlockSpec((_BLK, 1), lambda i: (i, 0)),
            pl.BlockSpec((G, 16), lambda i: (0, 0)),
            pl.BlockSpec((F + 16, MLP_HID), lambda i: (0, 0)),
            pl.BlockSpec((1, MLP_HID), lambda i: (0, 0)),
            pl.BlockSpec((MLP_HID, MLP_HID), lambda i: (0, 0)),
            pl.BlockSpec((1, MLP_HID), lambda i: (0, 0)),
            pl.BlockSpec((MLP_HID, 2), lambda i: (0, 0)),
            pl.BlockSpec((1, 2), lambda i: (0, 0)),
        ],
        out_specs=pl.BlockSpec((G, 2), lambda i: (0, 0)),
        out_shape=jax.ShapeDtypeStruct((G, 2), jnp.float32),
        scratch_shapes=[pltpu.VMEM((G, F), jnp.float32)],
    )(p, dd, b3, seg, gf, M1, Mb1, M2, Mb2, M3, Mb3)


def kernel(x, edge_index, node_graph_ids, global_feats, W1, b1, W2, b2,
           W3, b3, M1, Mb1, M2, Mb2, M3, Mb3):
    src = edge_index[0].astype(jnp.int32)
    dst = edge_index[1].astype(jnp.int32)
    seg = node_graph_ids.astype(jnp.int32).reshape(N, 1)

    npad = E_PAD - E
    pad_sink = N + (jnp.arange(npad, dtype=jnp.int32) % PADR)
    srcg = jnp.concatenate([src, jnp.zeros((npad,), jnp.int32)])
    srcg = srcg.reshape(IDX_ROWS, CH)
    dstg = jnp.concatenate([dst, pad_sink]).reshape(IDX_ROWS, CH)
    srcd = jnp.concatenate([src, pad_sink]).reshape(IDX_ROWS, CH)
    idx2 = jnp.stack([srcd, dstg])

    zeros64 = jnp.zeros((N, FH), jnp.float32)
    ones64 = jnp.ones((CH, FH), jnp.float32)

    degs = _sc_degrees(idx2, zeros64, ones64)
    ds, dd = degs[0], degs[1]

    hw = _tc_pre1(x, ds, W1)
    p = _sc_edge_pass(hw, srcg, dstg, zeros64)
    hw = _tc_pre23(p, dd, b1.reshape(1, F), ds, W2)
    p = _sc_edge_pass(hw, srcg, dstg, zeros64)
    hw = _tc_pre23(p, dd, b2.reshape(1, F), ds, W3)
    p = _sc_edge_pass(hw, srcg, dstg, zeros64)

    return _tc_head(p, dd, b3.reshape(1, F), seg, global_feats, M1,
                    Mb1.reshape(1, MLP_HID), M2, Mb2.reshape(1, MLP_HID),
                    M3, Mb3.reshape(1, 2))


# speedup vs baseline: 6.7584x; 1.0964x over previous
import jax
import jax.numpy as jnp
from jax import lax
from jax.experimental import pallas as pl
from jax.experimental.pallas import tpu as pltpu
from jax.experimental.pallas import tpu_sc as plsc

N = 10000
E = 320000
G = 64
F = 128
FH = F // 2
MLP_HID = 1024

NC, NS = 2, 16
CH = 128
RPSUB = 160
IDXB = 16
IDX_ROWS = NS * RPSUB
E_PAD = IDX_ROWS * CH
PADR = 512
ACC_ROWS = N + PADR
RPS = 632
RPS_LAST = N - (NS - 1) * RPS

_HIGH = jax.lax.Precision.HIGHEST

_mesh_cache = []


def _vec_mesh():
    if not _mesh_cache:
        _mesh_cache.append(plsc.VectorSubcoreMesh(
            core_axis_name="c", subcore_axis_name="s",
            num_cores=NC, num_subcores=NS))
    return _mesh_cache[0]


def _per_sub_copy(sid, mk_src, mk_dst):
    @pl.when(sid < NS - 1)
    def _():
        start = pl.multiple_of(sid * RPS, 8)
        pltpu.sync_copy(mk_src(start, RPS), mk_dst(start, RPS))

    @pl.when(sid == NS - 1)
    def _():
        start = (NS - 1) * RPS
        pltpu.sync_copy(mk_src(start, RPS_LAST), mk_dst(start, RPS_LAST))


def _deg_body(idx2_hbm, zeros_hbm, ones_hbm, out_hbm, idxs, ones_v, acc):
    cid = lax.axis_index("c")
    sid = lax.axis_index("s")

    pltpu.sync_copy(ones_hbm, ones_v)
    _per_sub_copy(sid, lambda s, n: zeros_hbm.at[pl.ds(s, n)],
                  lambda s, n: acc.at[pl.ds(s, n)])
    plsc.subcore_barrier()

    base = pl.multiple_of(sid * RPSUB, 8)
    pltpu.sync_copy(idx2_hbm.at[cid, pl.ds(base, RPSUB)], idxs)

    @pl.loop(0, RPSUB)
    def _(j):
        pltpu.sync_copy(ones_v, acc.at[idxs.at[j]], add=True)

    plsc.subcore_barrier()
    _per_sub_copy(sid, lambda s, n: acc.at[pl.ds(s, n)],
                  lambda s, n: out_hbm.at[cid, pl.ds(s, n)])


def _sc_degrees(idx2, zeros64, ones64):
    k = pl.kernel(
        _deg_body,
        compiler_params=pltpu.CompilerParams(use_tc_tiling_on_sc=False),
        out_type=jax.ShapeDtypeStruct((NC, N, FH), jnp.float32),
        mesh=_vec_mesh(),
        scratch_types=[
            pltpu.VMEM((RPSUB, CH), jnp.int32),
            pltpu.VMEM((CH, FH), jnp.float32),
            pltpu.VMEM_SHARED((ACC_ROWS, FH), jnp.float32),
        ],
    )
    return k(idx2, zeros64, ones64)


def _edge_body(hw2_hbm, sidx_hbm, didx_hbm, zeros_hbm, out_hbm, sidx, didx,
               b0, b1, b2, b3, sem0, sem1, sem2, sem3, sem4, sem5, sem6,
               sem7, table, acc):
    cid = lax.axis_index("c")
    sid = lax.axis_index("s")

    _per_sub_copy(sid, lambda s, n: hw2_hbm.at[cid, pl.ds(s, n)],
                  lambda s, n: table.at[pl.ds(s, n)])
    _per_sub_copy(sid, lambda s, n: zeros_hbm.at[pl.ds(s, n)],
                  lambda s, n: acc.at[pl.ds(s, n)])
    plsc.subcore_barrier()

    bufs = (b0, b1, b2, b3)
    gsems = (sem0, sem1, sem2, sem3)
    ssems = (sem4, sem5, sem6, sem7)

    @pl.loop(0, RPSUB // IDXB)
    def _(k):
        off = pl.multiple_of(sid * RPSUB + k * IDXB, 8)
        pltpu.sync_copy(sidx_hbm.at[pl.ds(off, IDXB)], sidx)
        pltpu.sync_copy(didx_hbm.at[pl.ds(off, IDXB)], didx)

        for u in range(4):
            pltpu.async_copy(table.at[sidx.at[u]], bufs[u], gsems[u])

        @pl.loop(0, IDXB // 4)
        def _(g):
            j0 = g * 4
            for u in range(4):
                pltpu.make_async_copy(
                    table.at[sidx.at[j0 + u]], bufs[u], gsems[u]).wait()
                pltpu.async_copy(bufs[u], acc.at[didx.at[j0 + u]],
                                 ssems[u], add=True)
            for u in range(4):
                pltpu.make_async_copy(bufs[u], acc.at[didx.at[j0 + u]],
                                      ssems[u]).wait()

                @pl.when(g < IDXB // 4 - 1)
                def _():
                    pltpu.async_copy(table.at[sidx.at[j0 + 4 + u]],
                                     bufs[u], gsems[u])

    plsc.subcore_barrier()
    _per_sub_copy(sid, lambda s, n: acc.at[pl.ds(s, n)],
                  lambda s, n: out_hbm.at[cid, pl.ds(s, n)])


def _sc_edge_pass(hw2, srcg, dstg, zeros64):
    k = pl.kernel(
        _edge_body,
        compiler_params=pltpu.CompilerParams(use_tc_tiling_on_sc=False),
        out_type=jax.ShapeDtypeStruct((NC, N, FH), jnp.float32),
        mesh=_vec_mesh(),
        scratch_types=[
            pltpu.VMEM((IDXB, CH), jnp.int32),
            pltpu.VMEM((IDXB, CH), jnp.int32),
            pltpu.VMEM((CH, FH), jnp.float32),
            pltpu.VMEM((CH, FH), jnp.float32),
            pltpu.VMEM((CH, FH), jnp.float32),
            pltpu.VMEM((CH, FH), jnp.float32),
            pltpu.SemaphoreType.DMA,
            pltpu.SemaphoreType.DMA,
            pltpu.SemaphoreType.DMA,
            pltpu.SemaphoreType.DMA,
            pltpu.SemaphoreType.DMA,
            pltpu.SemaphoreType.DMA,
            pltpu.SemaphoreType.DMA,
            pltpu.SemaphoreType.DMA,
            pltpu.VMEM_SHARED((N, FH), jnp.float32),
            pltpu.VMEM_SHARED((ACC_ROWS, FH), jnp.float32),
        ],
    )
    return k(hw2, srcg, dstg, zeros64)


_BLK = 2000
_NBLK = N // _BLK


def _pre1_body(x_ref, ds_ref, w_ref, o_ref):
    ns = lax.rsqrt(jnp.maximum(ds_ref[:, 0:1], 1.0))
    y = jnp.dot(x_ref[...], w_ref[...], precision=_HIGH) * ns
    o_ref[0] = y[:, :FH]
    o_ref[1] = y[:, FH:]


def _tc_pre1(x, ds, W1):
    return pl.pallas_call(
        _pre1_body,
        grid=(_NBLK,),
        in_specs=[
            pl.BlockSpec((_BLK, F), lambda i: (i, 0)),
            pl.BlockSpec((_BLK, FH), lambda i: (i, 0)),
            pl.BlockSpec((F, F), lambda i: (0, 0)),
        ],
        out_specs=pl.BlockSpec((NC, _BLK, FH), lambda i: (0, i, 0)),
        out_shape=jax.ShapeDtypeStruct((NC, N, FH), jnp.float32),
    )(x, ds, W1)


def _pre23_body(p_ref, dd_ref, b_ref, ds_ref, w_ref, o_ref):
    nd = lax.rsqrt(jnp.maximum(dd_ref[:, 0:1], 1.0))
    agg = jnp.concatenate([p_ref[0], p_ref[1]], axis=-1)
    h = jnp.maximum(agg * nd + b_ref[...], 0.0)
    ns = lax.rsqrt(jnp.maximum(ds_ref[:, 0:1], 1.0))
    y = jnp.dot(h, w_ref[...], precision=_HIGH) * ns
    o_ref[0] = y[:, :FH]
    o_ref[1] = y[:, FH:]


def _tc_pre23(p, dd, b, ds, W):
    return pl.pallas_call(
        _pre23_body,
        grid=(_NBLK,),
        in_specs=[
            pl.BlockSpec((NC, _BLK, FH), lambda i: (0, i, 0)),
            pl.BlockSpec((_BLK, FH), lambda i: (i, 0)),
            pl.BlockSpec((1, F), lambda i: (0, 0)),
            pl.BlockSpec((_BLK, FH), lambda i: (i, 0)),
            pl.BlockSpec((F, F), lambda i: (0, 0)),
        ],
        out_specs=pl.BlockSpec((NC, _BLK, FH), lambda i: (0, i, 0)),
        out_shape=jax.ShapeDtypeStruct((NC, N, FH), jnp.float32),
    )(p, dd, b, ds, W)


def _head_body(p_ref, dd_ref, b_ref, seg_ref, gf_ref, m1_ref, mb1_ref,
               m2_ref, mb2_ref, m3_ref, mb3_ref, o_ref, acc_ref):
    i = pl.program_id(0)

    @pl.when(i == 0)
    def _():
        acc_ref[...] = jnp.zeros_like(acc_ref)

    nd = lax.rsqrt(jnp.maximum(dd_ref[:, 0:1], 1.0))
    agg = jnp.concatenate([p_ref[0], p_ref[1]], axis=-1)
    h = jnp.maximum(agg * nd + b_ref[...], 0.0)
    gids = lax.broadcasted_iota(jnp.int32, (_BLK, G), 1)
    oh = (seg_ref[...] == gids).astype(jnp.float32)
    acc_ref[...] += lax.dot_general(
        oh, h, (((0,), (0,)), ((), ())), precision=_HIGH)

    @pl.when(i == _NBLK - 1)
    def _():
        ge = acc_ref[...]
        z = jnp.dot(ge, m1_ref[0:F, :], precision=_HIGH)
        z += jnp.dot(gf_ref[...], m1_ref[F:F + 16, :], precision=_HIGH)
        z = jnp.maximum(z + mb1_ref[...], 0.0)
        z = jnp.maximum(
            jnp.dot(z, m2_ref[...], precision=_HIGH) + mb2_ref[...], 0.0)
        o_ref[...] = jnp.dot(z, m3_ref[...], precision=_HIGH) + mb3_ref[...]


def _tc_head(p, dd, b3, seg, gf, M1, Mb1, M2, Mb2, M3, Mb3):
    return pl.pallas_call(
        _head_body,
        grid=(_NBLK,),
        in_specs=[
            pl.BlockSpec((NC, _BLK, FH), lambda i: (0, i, 0)),
            pl.BlockSpec((_BLK, FH), lambda i: (i, 0)),
            pl.BlockSpec((1, F), lambda i: (0, 0)),
            pl.BlockSpec((_BLK, 1), lambda i: (i, 0)),
            pl.BlockSpec((G, 16), lambda i: (0, 0)),
            pl.BlockSpec((F + 16, MLP_HID), lambda i: (0, 0)),
            pl.BlockSpec((1, MLP_HID), lambda i: (0, 0)),
            pl.BlockSpec((MLP_HID, MLP_HID), lambda i: (0, 0)),
            pl.BlockSpec((1, MLP_HID), lambda i: (0, 0)),
            pl.BlockSpec((MLP_HID, 2), lambda i: (0, 0)),
            pl.BlockSpec((1, 2), lambda i: (0, 0)),
        ],
        out_specs=pl.BlockSpec((G, 2), lambda i: (0, 0)),
        out_shape=jax.ShapeDtypeStruct((G, 2), jnp.float32),
        scratch_shapes=[pltpu.VMEM((G, F), jnp.float32)],
    )(p, dd, b3, seg, gf, M1, Mb1, M2, Mb2, M3, Mb3)


def kernel(x, edge_index, node_graph_ids, global_feats, W1, b1, W2, b2,
           W3, b3, M1, Mb1, M2, Mb2, M3, Mb3):
    src = edge_index[0].astype(jnp.int32)
    dst = edge_index[1].astype(jnp.int32)
    seg = node_graph_ids.astype(jnp.int32).reshape(N, 1)

    npad = E_PAD - E
    pad_sink = N + (jnp.arange(npad, dtype=jnp.int32) % PADR)
    srcg = jnp.concatenate([src, jnp.zeros((npad,), jnp.int32)])
    srcg = srcg.reshape(IDX_ROWS, CH)
    dstg = jnp.concatenate([dst, pad_sink]).reshape(IDX_ROWS, CH)
    srcd = jnp.concatenate([src, pad_sink]).reshape(IDX_ROWS, CH)
    idx2 = jnp.stack([srcd, dstg])

    zeros64 = jnp.zeros((N, FH), jnp.float32)
    ones64 = jnp.ones((CH, FH), jnp.float32)

    degs = _sc_degrees(idx2, zeros64, ones64)
    ds, dd = degs[0], degs[1]

    hw = _tc_pre1(x, ds, W1)
    p = _sc_edge_pass(hw, srcg, dstg, zeros64)
    hw = _tc_pre23(p, dd, b1.reshape(1, F), ds, W2)
    p = _sc_edge_pass(hw, srcg, dstg, zeros64)
    hw = _tc_pre23(p, dd, b2.reshape(1, F), ds, W3)
    p = _sc_edge_pass(hw, srcg, dstg, zeros64)

    return _tc_head(p, dd, b3.reshape(1, F), seg, global_feats, M1,
                    Mb1.reshape(1, MLP_HID), M2, Mb2.reshape(1, MLP_HID),
                    M3, Mb3.reshape(1, 2))


# speedup vs baseline: 6.7612x; 1.0004x over previous
import jax
import jax.numpy as jnp
from jax import lax
from jax.experimental import pallas as pl
from jax.experimental.pallas import tpu as pltpu
from jax.experimental.pallas import tpu_sc as plsc

N = 10000
E = 320000
G = 64
F = 128
FH = F // 2
MLP_HID = 1024

NC, NS = 2, 16
CH = 128
RPSUB = 160
IDXB = 16
IDX_ROWS = NS * RPSUB
E_PAD = IDX_ROWS * CH
PADR = 512
ACC_ROWS = N + PADR
RPS = 632
RPS_LAST = N - (NS - 1) * RPS

_HIGH = jax.lax.Precision.HIGHEST

_mesh_cache = []


def _vec_mesh():
    if not _mesh_cache:
        _mesh_cache.append(plsc.VectorSubcoreMesh(
            core_axis_name="c", subcore_axis_name="s",
            num_cores=NC, num_subcores=NS))
    return _mesh_cache[0]


def _per_sub_copy(sid, mk_src, mk_dst):
    @pl.when(sid < NS - 1)
    def _():
        start = pl.multiple_of(sid * RPS, 8)
        pltpu.sync_copy(mk_src(start, RPS), mk_dst(start, RPS))

    @pl.when(sid == NS - 1)
    def _():
        start = (NS - 1) * RPS
        pltpu.sync_copy(mk_src(start, RPS_LAST), mk_dst(start, RPS_LAST))


def _deg_body(idx2_hbm, zeros_hbm, ones_hbm, out_hbm, idxs, ones_v, acc):
    cid = lax.axis_index("c")
    sid = lax.axis_index("s")

    pltpu.sync_copy(ones_hbm, ones_v)
    _per_sub_copy(sid, lambda s, n: zeros_hbm.at[pl.ds(s, n)],
                  lambda s, n: acc.at[pl.ds(s, n)])
    plsc.subcore_barrier()

    base = pl.multiple_of(sid * RPSUB, 8)
    pltpu.sync_copy(idx2_hbm.at[cid, pl.ds(base, RPSUB)], idxs)

    @pl.loop(0, RPSUB)
    def _(j):
        pltpu.sync_copy(ones_v, acc.at[idxs.at[j]], add=True)

    plsc.subcore_barrier()
    _per_sub_copy(sid, lambda s, n: acc.at[pl.ds(s, n)],
                  lambda s, n: out_hbm.at[cid, pl.ds(s, n)])


def _sc_degrees(idx2, zeros64, ones64):
    k = pl.kernel(
        _deg_body,
        compiler_params=pltpu.CompilerParams(use_tc_tiling_on_sc=False),
        out_type=jax.ShapeDtypeStruct((NC, N, FH), jnp.float32),
        mesh=_vec_mesh(),
        scratch_types=[
            pltpu.VMEM((RPSUB, CH), jnp.int32),
            pltpu.VMEM((CH, FH), jnp.float32),
            pltpu.VMEM_SHARED((ACC_ROWS, FH), jnp.float32),
        ],
    )
    return k(idx2, zeros64, ones64)


def _edge_body(hw2_hbm, sidx_hbm, didx_hbm, zeros_hbm, out_hbm, sidx, didx,
               b0, b1, b2, b3, sem0, sem1, sem2, sem3, sem4, sem5, sem6,
               sem7, table, acc):
    cid = lax.axis_index("c")
    sid = lax.axis_index("s")

    _per_sub_copy(sid, lambda s, n: hw2_hbm.at[cid, pl.ds(s, n)],
                  lambda s, n: table.at[pl.ds(s, n)])
    _per_sub_copy(sid, lambda s, n: zeros_hbm.at[pl.ds(s, n)],
                  lambda s, n: acc.at[pl.ds(s, n)])
    plsc.subcore_barrier()

    bufs = (b0, b1, b2, b3)
    gsems = (sem0, sem1, sem2, sem3)
    ssems = (sem4, sem5, sem6, sem7)

    @pl.loop(0, RPSUB // IDXB)
    def _(k):
        off = pl.multiple_of(sid * RPSUB + k * IDXB, 8)
        pltpu.sync_copy(sidx_hbm.at[pl.ds(off, IDXB)], sidx)
        pltpu.sync_copy(didx_hbm.at[pl.ds(off, IDXB)], didx)

        for u in range(4):
            pltpu.async_copy(table.at[sidx.at[u]], bufs[u], gsems[u])

        @pl.loop(0, IDXB // 4)
        def _(g):
            j0 = g * 4
            for u in range(4):
                pltpu.make_async_copy(
                    table.at[sidx.at[j0 + u]], bufs[u], gsems[u]).wait()
                pltpu.async_copy(bufs[u], acc.at[didx.at[j0 + u]],
                                 ssems[u], add=True)
            for u in range(4):
                pltpu.make_async_copy(bufs[u], acc.at[didx.at[j0 + u]],
                                      ssems[u]).wait()

                @pl.when(g < IDXB // 4 - 1)
                def _():
                    pltpu.async_copy(table.at[sidx.at[j0 + 4 + u]],
                                     bufs[u], gsems[u])

    plsc.subcore_barrier()
    _per_sub_copy(sid, lambda s, n: acc.at[pl.ds(s, n)],
                  lambda s, n: out_hbm.at[cid, pl.ds(s, n)])


def _sc_edge_pass(hw2, srcg, dstg, zeros64):
    k = pl.kernel(
        _edge_body,
        compiler_params=pltpu.CompilerParams(use_tc_tiling_on_sc=False),
        out_type=jax.ShapeDtypeStruct((NC, N, FH), jnp.float32),
        mesh=_vec_mesh(),
        scratch_types=[
            pltpu.VMEM((IDXB, CH), jnp.int32),
            pltpu.VMEM((IDXB, CH), jnp.int32),
            pltpu.VMEM((CH, FH), jnp.float32),
            pltpu.VMEM((CH, FH), jnp.float32),
            pltpu.VMEM((CH, FH), jnp.float32),
            pltpu.VMEM((CH, FH), jnp.float32),
            pltpu.SemaphoreType.DMA,
            pltpu.SemaphoreType.DMA,
            pltpu.SemaphoreType.DMA,
            pltpu.SemaphoreType.DMA,
            pltpu.SemaphoreType.DMA,
            pltpu.SemaphoreType.DMA,
            pltpu.SemaphoreType.DMA,
            pltpu.SemaphoreType.DMA,
            pltpu.VMEM_SHARED((N, FH), jnp.float32),
            pltpu.VMEM_SHARED((ACC_ROWS, FH), jnp.float32),
        ],
    )
    return k(hw2, srcg, dstg, zeros64)


_BLK = 2000
_NBLK = N // _BLK


def _pre1_body(x_ref, ds_ref, w_ref, o_ref):
    ns = lax.rsqrt(jnp.maximum(ds_ref[:, 0:1], 1.0))
    y = jnp.dot(x_ref[...], w_ref[...]) * ns
    o_ref[0] = y[:, :FH]
    o_ref[1] = y[:, FH:]


def _tc_pre1(x, ds, W1):
    return pl.pallas_call(
        _pre1_body,
        grid=(_NBLK,),
        in_specs=[
            pl.BlockSpec((_BLK, F), lambda i: (i, 0)),
            pl.BlockSpec((_BLK, FH), lambda i: (i, 0)),
            pl.BlockSpec((F, F), lambda i: (0, 0)),
        ],
        out_specs=pl.BlockSpec((NC, _BLK, FH), lambda i: (0, i, 0)),
        out_shape=jax.ShapeDtypeStruct((NC, N, FH), jnp.float32),
    )(x, ds, W1)


def _pre23_body(p_ref, dd_ref, b_ref, ds_ref, w_ref, o_ref):
    nd = lax.rsqrt(jnp.maximum(dd_ref[:, 0:1], 1.0))
    agg = jnp.concatenate([p_ref[0], p_ref[1]], axis=-1)
    h = jnp.maximum(agg * nd + b_ref[...], 0.0)
    ns = lax.rsqrt(jnp.maximum(ds_ref[:, 0:1], 1.0))
    y = jnp.dot(h, w_ref[...]) * ns
    o_ref[0] = y[:, :FH]
    o_ref[1] = y[:, FH:]


def _tc_pre23(p, dd, b, ds, W):
    return pl.pallas_call(
        _pre23_body,
        grid=(_NBLK,),
        in_specs=[
            pl.BlockSpec((NC, _BLK, FH), lambda i: (0, i, 0)),
            pl.BlockSpec((_BLK, FH), lambda i: (i, 0)),
            pl.BlockSpec((1, F), lambda i: (0, 0)),
            pl.BlockSpec((_BLK, FH), lambda i: (i, 0)),
            pl.BlockSpec((F, F), lambda i: (0, 0)),
        ],
        out_specs=pl.BlockSpec((NC, _BLK, FH), lambda i: (0, i, 0)),
        out_shape=jax.ShapeDtypeStruct((NC, N, FH), jnp.float32),
    )(p, dd, b, ds, W)


def _head_body(p_ref, dd_ref, b_ref, seg_ref, gf_ref, m1_ref, mb1_ref,
               m2_ref, mb2_ref, m3_ref, mb3_ref, o_ref, acc_ref):
    i = pl.program_id(0)

    @pl.when(i == 0)
    def _():
        acc_ref[...] = jnp.zeros_like(acc_ref)

    nd = lax.rsqrt(jnp.maximum(dd_ref[:, 0:1], 1.0))
    agg = jnp.concatenate([p_ref[0], p_ref[1]], axis=-1)
    h = jnp.maximum(agg * nd + b_ref[...], 0.0)
    gids = lax.broadcasted_iota(jnp.int32, (_BLK, G), 1)
    oh = (seg_ref[...] == gids).astype(jnp.float32)
    acc_ref[...] += lax.dot_general(
        oh, h, (((0,), (0,)), ((), ())), precision=_HIGH)

    @pl.when(i == _NBLK - 1)
    def _():
        ge = acc_ref[...]
        z = jnp.dot(ge, m1_ref[0:F, :])
        z += jnp.dot(gf_ref[...], m1_ref[F:F + 16, :])
        z = jnp.maximum(z + mb1_ref[...], 0.0)
        z = jnp.maximum(
            jnp.dot(z, m2_ref[...]) + mb2_ref[...], 0.0)
        o_ref[...] = jnp.dot(z, m3_ref[...]) + mb3_ref[...]


def _tc_head(p, dd, b3, seg, gf, M1, Mb1, M2, Mb2, M3, Mb3):
    return pl.pallas_call(
        _head_body,
        grid=(_NBLK,),
        in_specs=[
            pl.BlockSpec((NC, _BLK, FH), lambda i: (0, i, 0)),
            pl.BlockSpec((_BLK, FH), lambda i: (i, 0)),
            pl.BlockSpec((1, F), lambda i: (0, 0)),
            pl.BlockSpec((_BLK, 1), lambda i: (i, 0)),
            pl.BlockSpec((G, 16), lambda i: (0, 0)),
            pl.BlockSpec((F + 16, MLP_HID), lambda i: (0, 0)),
            pl.BlockSpec((1, MLP_HID), lambda i: (0, 0)),
            pl.BlockSpec((MLP_HID, MLP_HID), lambda i: (0, 0)),
            pl.BlockSpec((1, MLP_HID), lambda i: (0, 0)),
            pl.BlockSpec((MLP_HID, 2), lambda i: (0, 0)),
            pl.BlockSpec((1, 2), lambda i: (0, 0)),
        ],
        out_specs=pl.BlockSpec((G, 2), lambda i: (0, 0)),
        out_shape=jax.ShapeDtypeStruct((G, 2), jnp.float32),
        scratch_shapes=[pltpu.VMEM((G, F), jnp.float32)],
    )(p, dd, b3, seg, gf, M1, Mb1, M2, Mb2, M3, Mb3)


def kernel(x, edge_index, node_graph_ids, global_feats, W1, b1, W2, b2,
           W3, b3, M1, Mb1, M2, Mb2, M3, Mb3):
    src = edge_index[0].astype(jnp.int32)
    dst = edge_index[1].astype(jnp.int32)
    seg = node_graph_ids.astype(jnp.int32).reshape(N, 1)

    npad = E_PAD - E
    pad_sink = N + (jnp.arange(npad, dtype=jnp.int32) % PADR)
    srcg = jnp.concatenate([src, jnp.zeros((npad,), jnp.int32)])
    srcg = srcg.reshape(IDX_ROWS, CH)
    dstg = jnp.concatenate([dst, pad_sink]).reshape(IDX_ROWS, CH)
    srcd = jnp.concatenate([src, pad_sink]).reshape(IDX_ROWS, CH)
    idx2 = jnp.stack([srcd, dstg])

    zeros64 = jnp.zeros((N, FH), jnp.float32)
    ones64 = jnp.ones((CH, FH), jnp.float32)

    degs = _sc_degrees(idx2, zeros64, ones64)
    ds, dd = degs[0], degs[1]

    hw = _tc_pre1(x, ds, W1)
    p = _sc_edge_pass(hw, srcg, dstg, zeros64)
    hw = _tc_pre23(p, dd, b1.reshape(1, F), ds, W2)
    p = _sc_edge_pass(hw, srcg, dstg, zeros64)
    hw = _tc_pre23(p, dd, b2.reshape(1, F), ds, W3)
    p = _sc_edge_pass(hw, srcg, dstg, zeros64)

    return _tc_head(p, dd, b3.reshape(1, F), seg, global_feats, M1,
                    Mb1.reshape(1, MLP_HID), M2, Mb2.reshape(1, MLP_HID),
                    M3, Mb3.reshape(1, 2))
